# baseline (speedup 1.0000x reference)
import functools

import jax
import jax.numpy as jnp
from jax.experimental import pallas as pl
from jax.experimental.pallas import tpu as pltpu

BATCH = 128
KEEP = 96
R, C = 512, 512


def _copy_body(idx_ref, x_ref, o_ref):
    o_ref[...] = x_ref[...]


@functools.partial(jax.jit, static_argnames=())
def _tc_gather(x, idx):
    grid_spec = pltpu.PrefetchScalarGridSpec(
        num_scalar_prefetch=1,
        grid=(KEEP,),
        in_specs=[
            pl.BlockSpec((1, R, C), lambda i, idx_ref: (idx_ref[i], 0, 0)),
        ],
        out_specs=pl.BlockSpec((1, R, C), lambda i, idx_ref: (i, 0, 0)),
    )
    return pl.pallas_call(
        _copy_body,
        grid_spec=grid_spec,
        out_shape=jax.ShapeDtypeStruct((KEEP, R, C), jnp.float32),
    )(idx, x)


def kernel(inputs):
    perm = jax.random.permutation(jax.random.key(0), BATCH).astype(jnp.int32)
    out = _tc_gather(inputs, perm[:KEEP])
    return out, perm


# speedup vs baseline: 1.0754x; 1.0754x over previous
import functools

import jax
import jax.numpy as jnp
from jax import lax
from jax.experimental import pallas as pl
from jax.experimental.pallas import tpu as pltpu
from jax.experimental.pallas import tpu_sc as plsc

BATCH = 128
KEEP = 96
R, C = 512, 512

NC, NS = 2, 16
NW = NC * NS
ROWS_PER_W = KEEP // NW
CHUNK_R = 64
NCHUNKS = R // CHUNK_R
NTASKS = ROWS_PER_W * NCHUNKS
NBUF = 3


@functools.partial(
    pl.kernel,
    out_type=(
        jax.ShapeDtypeStruct((KEEP, R, C), jnp.float32),
        jax.ShapeDtypeStruct((BATCH,), jnp.int32),
    ),
    mesh=plsc.VectorSubcoreMesh(core_axis_name="c", subcore_axis_name="s"),
    scratch_types=[
        pltpu.VMEM((16,), jnp.int32),
        pltpu.VMEM((BATCH,), jnp.int32),
        pltpu.VMEM((CHUNK_R, C), jnp.float32),
        pltpu.VMEM((CHUNK_R, C), jnp.float32),
        pltpu.VMEM((CHUNK_R, C), jnp.float32),
        pltpu.SemaphoreType.DMA,
        pltpu.SemaphoreType.DMA,
        pltpu.SemaphoreType.DMA,
        pltpu.SemaphoreType.DMA,
        pltpu.SemaphoreType.DMA,
        pltpu.SemaphoreType.DMA,
    ],
)
def _sc_gather(x_hbm, idxmat_hbm, perm_hbm, out_hbm, idx_out_hbm,
               idx_v, perm_v, buf0, buf1, buf2,
               gsem0, gsem1, gsem2, ssem0, ssem1, ssem2):
    cid = lax.axis_index("c")
    sid = lax.axis_index("s")
    wid = sid * NC + cid

    @pl.when(wid == 0)
    def _():
        pltpu.sync_copy(perm_hbm, perm_v)
        pltpu.sync_copy(perm_v, idx_out_hbm)

    pltpu.sync_copy(idxmat_hbm.at[wid], idx_v)
    vec = idx_v[...]
    srcs = [vec[j] for j in range(ROWS_PER_W)]
    obase = wid * ROWS_PER_W

    bufs = (buf0, buf1, buf2)
    gsems = (gsem0, gsem1, gsem2)
    ssems = (ssem0, ssem1, ssem2)
    tasks = [(j, c) for j in range(ROWS_PER_W) for c in range(NCHUNKS)]

    def start_gather(k):
        j, c = tasks[k]
        p = k % NBUF
        return pltpu.async_copy(
            x_hbm.at[srcs[j], pl.ds(c * CHUNK_R, CHUNK_R), :],
            bufs[p], gsems[p],
        )

    def start_scatter(k):
        j, c = tasks[k]
        p = k % NBUF
        return pltpu.async_copy(
            bufs[p],
            out_hbm.at[obase + j, pl.ds(c * CHUNK_R, CHUNK_R), :],
            ssems[p],
        )

    G = NBUF - 1
    gathers = {k: start_gather(k) for k in range(G)}
    scatters = {}
    for k in range(NTASKS):
        gathers[k].wait()
        scatters[k] = start_scatter(k)
        if k + G < NTASKS:
            if k + G - NBUF >= 0:
                scatters[k + G - NBUF].wait()
            gathers[k + G] = start_gather(k + G)
    for k in range(max(0, NTASKS - NBUF), NTASKS):
        scatters[k].wait()


def kernel(inputs):
    perm = jax.random.permutation(jax.random.key(0), BATCH).astype(jnp.int32)
    idxmat = (
        jnp.zeros((NW, 16), jnp.int32)
        .at[:, :ROWS_PER_W]
        .set(perm[:KEEP].reshape(NW, ROWS_PER_W))
    )
    return _sc_gather(inputs, idxmat, perm)


# speedup vs baseline: 1.1110x; 1.0331x over previous
import functools

import jax
import jax.numpy as jnp
from jax import lax
from jax.experimental import pallas as pl
from jax.experimental.pallas import tpu as pltpu
from jax.experimental.pallas import tpu_sc as plsc

BATCH = 128
KEEP = 96
R, C = 512, 512

NC, NS = 2, 16
NW = NC * NS
ROWS_PER_W = KEEP // NW
CHUNK_R = 32
NCHUNKS = R // CHUNK_R
NBUF_A = 3
NBUF_B = 2


@functools.partial(
    pl.kernel,
    out_type=(
        jax.ShapeDtypeStruct((KEEP, R, C), jnp.float32),
        jax.ShapeDtypeStruct((BATCH,), jnp.int32),
    ),
    mesh=plsc.VectorSubcoreMesh(core_axis_name="c", subcore_axis_name="s"),
    scratch_types=[
        pltpu.VMEM((16,), jnp.int32),
        pltpu.VMEM((BATCH,), jnp.int32),
        pltpu.VMEM((CHUNK_R, C), jnp.float32),
        pltpu.VMEM((CHUNK_R, C), jnp.float32),
        pltpu.VMEM((CHUNK_R, C), jnp.float32),
        pltpu.VMEM_SHARED((NS * NBUF_B * CHUNK_R, C), jnp.float32),
        pltpu.SemaphoreType.DMA,
        pltpu.SemaphoreType.DMA,
        pltpu.SemaphoreType.DMA,
        pltpu.SemaphoreType.DMA,
        pltpu.SemaphoreType.DMA,
        pltpu.SemaphoreType.DMA,
        pltpu.SemaphoreType.DMA,
        pltpu.SemaphoreType.DMA,
        pltpu.SemaphoreType.DMA,
        pltpu.SemaphoreType.DMA,
    ],
)
def _sc_gather(x_hbm, idxmat_hbm, perm_hbm, out_hbm, idx_out_hbm,
               idx_v, perm_v, bufa0, bufa1, bufa2, ringb,
               ga0, ga1, ga2, sa0, sa1, sa2, gb0, gb1, sb0, sb1):
    cid = lax.axis_index("c")
    sid = lax.axis_index("s")
    wid = sid * NC + cid

    @pl.when(wid == 0)
    def _():
        pltpu.sync_copy(perm_hbm, perm_v)
        pltpu.sync_copy(perm_v, idx_out_hbm)

    pltpu.sync_copy(idxmat_hbm.at[wid], idx_v)
    vec = idx_v[...]
    srcs = [vec[j] for j in range(ROWS_PER_W)]
    obase = wid * ROWS_PER_W

    all_tasks = [(j, c) for j in range(ROWS_PER_W) for c in range(NCHUNKS)]
    bufs_a = (bufa0, bufa1, bufa2)
    bufs_b = tuple(
        ringb.at[pl.ds((sid * NBUF_B + p) * CHUNK_R, CHUNK_R), :]
        for p in range(NBUF_B)
    )

    paths = [
        dict(tasks=all_tasks[0::2], bufs=bufs_a, nbuf=NBUF_A,
             gsems=(ga0, ga1, ga2), ssems=(sa0, sa1, sa2)),
        dict(tasks=all_tasks[1::2], bufs=bufs_b, nbuf=NBUF_B,
             gsems=(gb0, gb1), ssems=(sb0, sb1)),
    ]

    def start_gather(P, k):
        j, c = P["tasks"][k]
        p = k % P["nbuf"]
        return pltpu.async_copy(
            x_hbm.at[srcs[j], pl.ds(c * CHUNK_R, CHUNK_R), :],
            P["bufs"][p], P["gsems"][p],
        )

    def start_scatter(P, k):
        j, c = P["tasks"][k]
        p = k % P["nbuf"]
        return pltpu.async_copy(
            P["bufs"][p],
            out_hbm.at[obase + j, pl.ds(c * CHUNK_R, CHUNK_R), :],
            P["ssems"][p],
        )

    for P in paths:
        P["G"] = P["nbuf"] - 1
        P["gathers"] = {k: start_gather(P, k) for k in range(P["G"])}
        P["scatters"] = {}
    n = len(paths[0]["tasks"])
    assert all(len(P["tasks"]) == n for P in paths)
    for k in range(n):
        for P in paths:
            g, s, G, nbuf = P["gathers"], P["scatters"], P["G"], P["nbuf"]
            g[k].wait()
            s[k] = start_scatter(P, k)
            if k + G < n:
                if k + G - nbuf >= 0:
                    s[k + G - nbuf].wait()
                g[k + G] = start_gather(P, k + G)
    for P in paths:
        for k in range(max(0, n - P["nbuf"]), n):
            P["scatters"][k].wait()


def kernel(inputs):
    perm = jax.random.permutation(jax.random.key(0), BATCH).astype(jnp.int32)
    idxmat = (
        jnp.zeros((NW, 16), jnp.int32)
        .at[:, :ROWS_PER_W]
        .set(perm[:KEEP].reshape(NW, ROWS_PER_W))
    )
    return _sc_gather(inputs, idxmat, perm)
